# Initial kernel scaffold; baseline (speedup 1.0000x reference)
import jax
import jax.numpy as jnp
from jax.experimental import pallas as pl


def kernel(wind_direction, wind_speed, yaw, layout, node_in_W, node_in_b, edge_in_W, edge_in_b, msg_W1, msg_b1, msg_W2, msg_b2, upd_W1, upd_b1, upd_W2, upd_b2, att_W, att_b, edge_index):
    raise NotImplementedError("write your pallas kernel here")


# speedup vs baseline: 15.8643x; 15.8643x over previous
import jax
import jax.numpy as jnp
from jax.experimental import pallas as pl
from jax.experimental.pallas import tpu as pltpu

_B, _N, _NODE_DIM, _EDGE_DIM, _N_LAYERS = 32, 100, 64, 16, 4
_E = _N * _N


def _silu(x):
    return x * jax.nn.sigmoid(x)


def _mm(a, b):
    return jax.lax.dot_general(
        a, b, (((1,), (0,)), ((), ())), preferred_element_type=jnp.float32
    )


def _gnn_kernel(raw_ref, R_ref, T_ref, Rt_ref, mask_ref,
                niWt_ref, nib_ref, eiWt_ref, eib_ref,
                mW1t_ref, mb1_ref, mW2t_ref, mb2_ref,
                uW1t_ref, ub1_ref, uW2t_ref, ub2_ref,
                aWt_ref, ab_ref, out_ref):
    raw = raw_ref[0]
    wd = raw[0:1] * (jnp.pi / 180.0)
    ws = raw[1:2] * (1.0 / 28.0)
    yw = raw[2:3] * (jnp.pi / 180.0)
    lx = raw[3:4] * 2.0 - 1.0
    ly = raw[4:5] * 2.0 - 1.0
    wx = ws * jnp.cos(wd)
    wy = ws * jnp.sin(wd)

    R = R_ref[...]
    T = T_ref[...]
    Rt = Rt_ref[...]
    mask = mask_ref[...]

    S = jnp.concatenate([ws, yw, lx, ly, wx, wy], axis=0)
    Ss = _mm(S, R)
    D = jnp.concatenate([lx, ly], axis=0)
    Dd = _mm(D, T)

    dx = Dd[0:1] - Ss[2:3]
    dy = Dd[1:2] - Ss[3:4]
    radial = jnp.sqrt(dx * dx + dy * dy)
    wdot = Ss[4:5] * dx + Ss[5:6] * dy
    wcross = Ss[4:5] * dy - Ss[5:6] * dx
    ef = jnp.concatenate([radial, Ss[0:1], wdot, wcross, Ss[1:2]], axis=0)

    e = _mm(eiWt_ref[...], ef) + eib_ref[...]
    seg = _mm(ef * mask, Rt)
    nf = jnp.concatenate([ws, seg], axis=0)
    h = _mm(niWt_ref[...], nf) + nib_ref[...]

    aWt = aWt_ref[...]
    ab = ab_ref[...]
    for i in range(_N_LAYERS):
        W1t = mW1t_ref[i]
        hs = _mm(W1t[:, 0:64], h)
        hd = _mm(W1t[:, 64:128], h)
        pre = _mm(hs, R) + _mm(hd, T) + _mm(W1t[:, 128:144], e) + mb1_ref[i]
        m = _silu(pre)
        m = _silu(_mm(mW2t_ref[i], m) + mb2_ref[i])
        m = m * jax.nn.sigmoid(_mm(aWt, m) + ab)
        aggr = _mm(m * mask, Rt)
        u = jnp.concatenate([h, aggr], axis=0)
        u = _silu(_mm(uW1t_ref[i], u) + ub1_ref[i])
        u = _silu(_mm(uW2t_ref[i], u) + ub2_ref[i])
        h = h + u
        if i < _N_LAYERS - 1:
            e = e + m
    out_ref[0] = h


def kernel(wind_direction, wind_speed, yaw, layout, node_in_W, node_in_b,
           edge_in_W, edge_in_b, msg_W1, msg_b1, msg_W2, msg_b2,
           upd_W1, upd_b1, upd_W2, upd_b2, att_W, att_b, edge_index):
    f32 = jnp.float32
    raw = jnp.concatenate(
        [wind_direction, wind_speed, yaw, layout], axis=-1
    ).transpose(0, 2, 1).astype(f32)

    ii = jax.lax.broadcasted_iota(jnp.int32, (_N, _E), 0)
    ll = jax.lax.broadcasted_iota(jnp.int32, (_N, _E), 1)
    R = (ll // _N == ii).astype(f32)
    T = (ll % _N == ii).astype(f32)
    le = jax.lax.broadcasted_iota(jnp.int32, (_E, _N), 0)
    ie = jax.lax.broadcasted_iota(jnp.int32, (_E, _N), 1)
    Rt = (le // _N == ie).astype(f32)
    lane = jax.lax.broadcasted_iota(jnp.int32, (1, _E), 1)
    mask = (lane // _N != lane % _N).astype(f32)

    niWt = node_in_W.T
    nib = node_in_b.reshape(_NODE_DIM, 1)
    eiWt = edge_in_W.T
    eib = edge_in_b.reshape(_EDGE_DIM, 1)
    mW1t = msg_W1.transpose(0, 2, 1)
    mb1 = msg_b1.reshape(_N_LAYERS, _EDGE_DIM, 1)
    mW2t = msg_W2.transpose(0, 2, 1)
    mb2 = msg_b2.reshape(_N_LAYERS, _EDGE_DIM, 1)
    uW1t = upd_W1.transpose(0, 2, 1)
    ub1 = upd_b1.reshape(_N_LAYERS, _NODE_DIM, 1)
    uW2t = upd_W2.transpose(0, 2, 1)
    ub2 = upd_b2.reshape(_N_LAYERS, _NODE_DIM, 1)
    aWt = att_W.T
    ab = att_b.reshape(1, 1)

    def full(x):
        return pl.BlockSpec(x.shape, lambda b: (0,) * x.ndim)

    out = pl.pallas_call(
        _gnn_kernel,
        grid=(_B,),
        in_specs=[
            pl.BlockSpec((1, 5, _N), lambda b: (b, 0, 0)),
            full(R), full(T), full(Rt), full(mask),
            full(niWt), full(nib), full(eiWt), full(eib),
            full(mW1t), full(mb1), full(mW2t), full(mb2),
            full(uW1t), full(ub1), full(uW2t), full(ub2),
            full(aWt), full(ab),
        ],
        out_specs=pl.BlockSpec((1, _NODE_DIM, _N), lambda b: (b, 0, 0)),
        out_shape=jax.ShapeDtypeStruct((_B, _NODE_DIM, _N), f32),
        compiler_params=pltpu.CompilerParams(
            dimension_semantics=("arbitrary",),
        ),
    )(raw, R, T, Rt, mask, niWt, nib, eiWt, eib,
      mW1t, mb1, mW2t, mb2, uW1t, ub1, uW2t, ub2, aWt, ab)
    return out.transpose(0, 2, 1)


# speedup vs baseline: 16.6110x; 1.0471x over previous
import jax
import jax.numpy as jnp
from jax.experimental import pallas as pl
from jax.experimental.pallas import tpu as pltpu

_B, _N, _NODE_DIM, _EDGE_DIM, _N_LAYERS = 32, 100, 64, 16, 4
_E = _N * _N
_BPG = 2


def _sigmoid(x):
    return 0.5 * jnp.tanh(0.5 * x) + 0.5


def _silu(x):
    return x * _sigmoid(x)


def _mm(a, b):
    return jax.lax.dot_general(
        a, b, (((1,), (0,)), ((), ())), preferred_element_type=jnp.float32
    )


def _one_batch(raw, R, T, Rt, mask, refs):
    (niWt_ref, nib_ref, eiWt_ref, eib_ref,
     mW1t_ref, mb1_ref, mW2t_ref, mb2_ref,
     uW1t_ref, ub1_ref, uW2t_ref, ub2_ref,
     aWt_ref, ab_ref) = refs
    wd = raw[0:1] * (jnp.pi / 180.0)
    ws = raw[1:2] * (1.0 / 28.0)
    yw = raw[2:3] * (jnp.pi / 180.0)
    lx = raw[3:4] * 2.0 - 1.0
    ly = raw[4:5] * 2.0 - 1.0
    wx = ws * jnp.cos(wd)
    wy = ws * jnp.sin(wd)

    S = jnp.concatenate([ws, yw, lx, ly, wx, wy], axis=0)
    Ss = _mm(S, R)
    D = jnp.concatenate([lx, ly], axis=0)
    Dd = _mm(D, T)

    dx = Dd[0:1] - Ss[2:3]
    dy = Dd[1:2] - Ss[3:4]
    radial = jnp.sqrt(dx * dx + dy * dy)
    wdot = Ss[4:5] * dx + Ss[5:6] * dy
    wcross = Ss[4:5] * dy - Ss[5:6] * dx
    ef = jnp.concatenate([radial, Ss[0:1], wdot, wcross, Ss[1:2]], axis=0)

    e = _mm(eiWt_ref[...], ef) + eib_ref[...]
    seg = _mm(ef * mask, Rt)
    niWt = niWt_ref[...]
    h = _mm(niWt[:, 0:1], ws) + _mm(niWt[:, 1:6], seg) + nib_ref[...]

    aWt = aWt_ref[...]
    ab = ab_ref[...]
    for i in range(_N_LAYERS):
        W1t = mW1t_ref[i]
        hs = _mm(W1t[:, 0:64], h)
        hd = _mm(W1t[:, 64:128], h)
        pre = _mm(hs, R) + _mm(hd, T) + _mm(W1t[:, 128:144], e) + mb1_ref[i]
        m = _silu(pre)
        m = _silu(_mm(mW2t_ref[i], m) + mb2_ref[i])
        m = m * _sigmoid(_mm(aWt, m) + ab)
        aggr = _mm(m * mask, Rt)
        uW1t = uW1t_ref[i]
        u = _silu(_mm(uW1t[:, 0:64], h) + _mm(uW1t[:, 64:80], aggr) + ub1_ref[i])
        u = _silu(_mm(uW2t_ref[i], u) + ub2_ref[i])
        h = h + u
        if i < _N_LAYERS - 1:
            e = e + m
    return h


def _gnn_kernel(raw_ref, R_ref, T_ref, Rt_ref, mask_ref,
                niWt_ref, nib_ref, eiWt_ref, eib_ref,
                mW1t_ref, mb1_ref, mW2t_ref, mb2_ref,
                uW1t_ref, ub1_ref, uW2t_ref, ub2_ref,
                aWt_ref, ab_ref, out_ref):
    R = R_ref[...]
    T = T_ref[...]
    Rt = Rt_ref[...]
    mask = mask_ref[...]
    refs = (niWt_ref, nib_ref, eiWt_ref, eib_ref,
            mW1t_ref, mb1_ref, mW2t_ref, mb2_ref,
            uW1t_ref, ub1_ref, uW2t_ref, ub2_ref,
            aWt_ref, ab_ref)
    for k in range(_BPG):
        out_ref[k] = _one_batch(raw_ref[k], R, T, Rt, mask, refs)


def kernel(wind_direction, wind_speed, yaw, layout, node_in_W, node_in_b,
           edge_in_W, edge_in_b, msg_W1, msg_b1, msg_W2, msg_b2,
           upd_W1, upd_b1, upd_W2, upd_b2, att_W, att_b, edge_index):
    f32 = jnp.float32
    raw = jnp.concatenate(
        [wind_direction, wind_speed, yaw, layout], axis=-1
    ).transpose(0, 2, 1).astype(f32)

    ii = jax.lax.broadcasted_iota(jnp.int32, (_N, _E), 0)
    ll = jax.lax.broadcasted_iota(jnp.int32, (_N, _E), 1)
    R = (ll // _N == ii).astype(f32)
    T = (ll % _N == ii).astype(f32)
    le = jax.lax.broadcasted_iota(jnp.int32, (_E, _N), 0)
    ie = jax.lax.broadcasted_iota(jnp.int32, (_E, _N), 1)
    Rt = (le // _N == ie).astype(f32)
    lane = jax.lax.broadcasted_iota(jnp.int32, (1, _E), 1)
    mask = (lane // _N != lane % _N).astype(f32)

    niWt = node_in_W.T
    nib = node_in_b.reshape(_NODE_DIM, 1)
    eiWt = edge_in_W.T
    eib = edge_in_b.reshape(_EDGE_DIM, 1)
    mW1t = msg_W1.transpose(0, 2, 1)
    mb1 = msg_b1.reshape(_N_LAYERS, _EDGE_DIM, 1)
    mW2t = msg_W2.transpose(0, 2, 1)
    mb2 = msg_b2.reshape(_N_LAYERS, _EDGE_DIM, 1)
    uW1t = upd_W1.transpose(0, 2, 1)
    ub1 = upd_b1.reshape(_N_LAYERS, _NODE_DIM, 1)
    uW2t = upd_W2.transpose(0, 2, 1)
    ub2 = upd_b2.reshape(_N_LAYERS, _NODE_DIM, 1)
    aWt = att_W.T
    ab = att_b.reshape(1, 1)

    def full(x):
        return pl.BlockSpec(x.shape, lambda b: (0,) * x.ndim)

    out = pl.pallas_call(
        _gnn_kernel,
        grid=(_B // _BPG,),
        in_specs=[
            pl.BlockSpec((_BPG, 5, _N), lambda b: (b, 0, 0)),
            full(R), full(T), full(Rt), full(mask),
            full(niWt), full(nib), full(eiWt), full(eib),
            full(mW1t), full(mb1), full(mW2t), full(mb2),
            full(uW1t), full(ub1), full(uW2t), full(ub2),
            full(aWt), full(ab),
        ],
        out_specs=pl.BlockSpec((_BPG, _NODE_DIM, _N), lambda b: (b, 0, 0)),
        out_shape=jax.ShapeDtypeStruct((_B, _NODE_DIM, _N), f32),
        compiler_params=pltpu.CompilerParams(
            dimension_semantics=("arbitrary",),
        ),
    )(raw, R, T, Rt, mask, niWt, nib, eiWt, eib,
      mW1t, mb1, mW2t, mb2, uW1t, ub1, uW2t, ub2, aWt, ab)
    return out.transpose(0, 2, 1)
